# baseline (speedup 1.0000x reference)
import functools

import jax
import jax.numpy as jnp
from jax import lax
from jax.experimental import pallas as pl
from jax.experimental.pallas import tpu as pltpu
from jax.experimental.pallas import tpu_sc as plsc

N_CAT = 238
D_IMG = 3 * 32 * 32
BM = 512


def _head_body(x_ref, w_ref, b_ref, u_ref, samp_ref, slp_ref, ent_ref, kl_ref):
    n_copies = u_ref.shape[0]
    x = x_ref[...]
    w = w_ref[...]
    logits = jnp.dot(x, w, preferred_element_type=jnp.float32) + b_ref[...]
    m = jnp.max(logits, axis=-1, keepdims=True)
    sh = logits - m
    lse = jnp.log(jnp.sum(jnp.exp(sh), axis=-1, keepdims=True))
    logp = sh - lse
    p = jnp.exp(logp)
    ent_ref[0, :] = -jnp.sum(p * logp, axis=-1)
    kl_ref[0, :] = jnp.sum(p * (logp - jnp.log(1.0 / N_CAT)), axis=-1)
    iota = lax.broadcasted_iota(jnp.int32, (BM, N_CAT), 1)
    for k in range(n_copies):
        g = -jnp.log(-jnp.log(u_ref[k]))
        s = jnp.argmax(logp + g, axis=-1).astype(jnp.int32)
        samp_ref[k, :] = s
        slp_ref[k, :] = jnp.sum(jnp.where(iota == s[:, None], logp, 0.0), axis=-1)


def _head_call(xf, W, b2, u):
    bsz = xf.shape[0]
    n_copies = u.shape[0]
    grid = (bsz // BM,)
    return pl.pallas_call(
        _head_body,
        grid=grid,
        in_specs=[
            pl.BlockSpec((BM, D_IMG), lambda i: (i, 0)),
            pl.BlockSpec((D_IMG, N_CAT), lambda i: (0, 0)),
            pl.BlockSpec((1, N_CAT), lambda i: (0, 0)),
            pl.BlockSpec((n_copies, BM, N_CAT), lambda i: (0, i, 0)),
        ],
        out_specs=[
            pl.BlockSpec((n_copies, BM), lambda i: (0, i)),
            pl.BlockSpec((n_copies, BM), lambda i: (0, i)),
            pl.BlockSpec((1, BM), lambda i: (0, i)),
            pl.BlockSpec((1, BM), lambda i: (0, i)),
        ],
        out_shape=[
            jax.ShapeDtypeStruct((n_copies, bsz), jnp.int32),
            jax.ShapeDtypeStruct((n_copies, bsz), jnp.float32),
            jax.ShapeDtypeStruct((1, bsz), jnp.float32),
            jax.ShapeDtypeStruct((1, bsz), jnp.float32),
        ],
        compiler_params=pltpu.CompilerParams(
            dimension_semantics=("parallel",),
        ),
    )(xf, W, b2, u)


_SC_CH = 32


def _sc_gather_body(per_w, idx_hbm, bank_hbm, out_hbm, idx_v, rows_v, sem):
    nc = 2
    wid = lax.axis_index("s") * nc + lax.axis_index("c")
    base = wid * per_w
    pltpu.sync_copy(idx_hbm.at[pl.ds(base, per_w)], idx_v)
    for c in range(per_w // _SC_CH):
        off = c * _SC_CH
        pltpu.async_copy(
            bank_hbm.at[idx_v.at[pl.ds(off, _SC_CH)]], rows_v, sem
        ).wait()
        pltpu.sync_copy(rows_v, out_hbm.at[pl.ds(base + off, _SC_CH)])


def _gather_call(idx, bank):
    n_rows = idx.shape[0]
    img_shape = bank.shape[1:]
    info = plsc.get_sparse_core_info()
    nw = info.num_cores * info.num_subcores
    per_w = n_rows // nw
    mesh = plsc.VectorSubcoreMesh(core_axis_name="c", subcore_axis_name="s")
    kfn = pl.kernel(
        functools.partial(_sc_gather_body, per_w),
        mesh=mesh,
        out_type=jax.ShapeDtypeStruct((n_rows,) + img_shape, jnp.float32),
        scratch_types=[
            pltpu.VMEM((per_w,), jnp.int32),
            pltpu.VMEM((_SC_CH,) + img_shape, jnp.float32),
            pltpu.SemaphoreType.DMA,
        ],
        compiler_params=pltpu.CompilerParams(use_tc_tiling_on_sc=False),
    )
    return kfn(idx, bank)


def kernel(x, u, W, b, bank, n_copies):
    bsz = x.shape[0]
    n_copies_static = u.shape[0]
    xf = x.reshape(bsz, -1)
    samp, slp, ent, kl = _head_call(xf, W, b.reshape(1, -1), u)
    idx = samp.reshape(-1)
    x_out = jax.lax.stop_gradient(_gather_call(idx, bank))
    return (x_out, slp.reshape(-1), ent.reshape(-1), kl.reshape(-1))


# speedup vs baseline: 1.9198x; 1.9198x over previous
import functools

import jax
import jax.numpy as jnp
from jax import lax
from jax.experimental import pallas as pl
from jax.experimental.pallas import tpu as pltpu
from jax.experimental.pallas import tpu_sc as plsc

N_CAT = 238
D_IMG = 3 * 32 * 32
BM = 512


def _head_body(x_ref, w_ref, b_ref, u_ref, samp_ref, slp_ref, ent_ref, kl_ref):
    n_copies = u_ref.shape[0]
    x = x_ref[...]
    w = w_ref[...]
    logits = jnp.dot(x, w, preferred_element_type=jnp.float32) + b_ref[...]
    m = jnp.max(logits, axis=-1, keepdims=True)
    sh = logits - m
    lse = jnp.log(jnp.sum(jnp.exp(sh), axis=-1, keepdims=True))
    logp = sh - lse
    p = jnp.exp(logp)
    ent_ref[0, :] = -jnp.sum(p * logp, axis=-1)
    kl_ref[0, :] = jnp.sum(p * (logp - jnp.log(1.0 / N_CAT)), axis=-1)
    iota = lax.broadcasted_iota(jnp.int32, (BM, N_CAT), 1)
    for k in range(n_copies):
        g = -jnp.log(-jnp.log(u_ref[k]))
        s = jnp.argmax(logp + g, axis=-1).astype(jnp.int32)
        samp_ref[k, :] = s
        slp_ref[k, :] = jnp.sum(jnp.where(iota == s[:, None], logp, 0.0), axis=-1)


def _head_call(xf, W, b2, u):
    bsz = xf.shape[0]
    n_copies = u.shape[0]
    grid = (bsz // BM,)
    return pl.pallas_call(
        _head_body,
        grid=grid,
        in_specs=[
            pl.BlockSpec((BM, D_IMG), lambda i: (i, 0)),
            pl.BlockSpec((D_IMG, N_CAT), lambda i: (0, 0)),
            pl.BlockSpec((1, N_CAT), lambda i: (0, 0)),
            pl.BlockSpec((n_copies, BM, N_CAT), lambda i: (0, i, 0)),
        ],
        out_specs=[
            pl.BlockSpec((n_copies, BM), lambda i: (0, i)),
            pl.BlockSpec((n_copies, BM), lambda i: (0, i)),
            pl.BlockSpec((1, BM), lambda i: (0, i)),
            pl.BlockSpec((1, BM), lambda i: (0, i)),
        ],
        out_shape=[
            jax.ShapeDtypeStruct((n_copies, bsz), jnp.int32),
            jax.ShapeDtypeStruct((n_copies, bsz), jnp.float32),
            jax.ShapeDtypeStruct((1, bsz), jnp.float32),
            jax.ShapeDtypeStruct((1, bsz), jnp.float32),
        ],
        compiler_params=pltpu.CompilerParams(
            dimension_semantics=("parallel",),
        ),
    )(xf, W, b2, u)


_SC_CH = 16


def _sc_gather_body(per_w, idx_hbm, bank_hbm, out_hbm,
                    idx_v, rows0, rows1, gs0, gs1, ss0, ss1):
    nc = 2
    wid = lax.axis_index("s") * nc + lax.axis_index("c")
    base = wid * per_w
    pltpu.sync_copy(idx_hbm.at[pl.ds(base, per_w)], idx_v)
    n_ch = per_w // _SC_CH
    bufs = ((rows0, gs0, ss0), (rows1, gs1, ss1))

    def gather(c, buf, gsem):
        pltpu.async_copy(
            bank_hbm.at[idx_v.at[pl.ds(c * _SC_CH, _SC_CH)]], buf, gsem
        )

    gather(0, rows0, gs0)
    if n_ch > 1:
        gather(1, rows1, gs1)
    for c in range(n_ch):
        buf, gsem, ssem = bufs[c % 2]
        pltpu.make_async_copy(
            bank_hbm.at[idx_v.at[pl.ds(0, _SC_CH)]], buf, gsem
        ).wait()
        st = pltpu.async_copy(
            buf, out_hbm.at[pl.ds(base + c * _SC_CH, _SC_CH)], ssem
        )
        st.wait()
        if c + 2 < n_ch:
            gather(c + 2, buf, gsem)


def _gather_call(idx, bank_flat):
    n_rows = idx.shape[0]
    info = plsc.get_sparse_core_info()
    nw = info.num_cores * info.num_subcores
    per_w = n_rows // nw
    mesh = plsc.VectorSubcoreMesh(core_axis_name="c", subcore_axis_name="s")
    kfn = pl.kernel(
        functools.partial(_sc_gather_body, per_w),
        mesh=mesh,
        out_type=jax.ShapeDtypeStruct((n_rows, D_IMG), jnp.float32),
        scratch_types=[
            pltpu.VMEM((per_w,), jnp.int32),
            pltpu.VMEM((_SC_CH, D_IMG), jnp.float32),
            pltpu.VMEM((_SC_CH, D_IMG), jnp.float32),
            pltpu.SemaphoreType.DMA,
            pltpu.SemaphoreType.DMA,
            pltpu.SemaphoreType.DMA,
            pltpu.SemaphoreType.DMA,
        ],
    )
    return kfn(idx, bank_flat)


def kernel(x, u, W, b, bank, n_copies):
    bsz = x.shape[0]
    n_copies_static = u.shape[0]
    xf = x.reshape(bsz, -1)
    samp, slp, ent, kl = _head_call(xf, W, b.reshape(1, -1), u)
    idx = samp.reshape(-1)
    bank_flat = bank.reshape(N_CAT, D_IMG)
    half = idx.shape[0] // 2
    rows_a = _gather_call(idx[:half], bank_flat)
    rows_b = _gather_call(idx[half:], bank_flat)
    img = bank.shape[1:]
    xo_a = rows_a.reshape((half,) + img)
    xo_b = rows_b.reshape((half,) + img)
    x_out = jax.lax.stop_gradient(jnp.concatenate([xo_a, xo_b], axis=0))
    return (x_out, slp.reshape(-1), ent.reshape(-1), kl.reshape(-1))


# speedup vs baseline: 5.8548x; 3.0497x over previous
import functools

import jax
import jax.numpy as jnp
from jax import lax
from jax.experimental import pallas as pl
from jax.experimental.pallas import tpu as pltpu
from jax.experimental.pallas import tpu_sc as plsc

N_CAT = 238
D_IMG = 3 * 32 * 32
BM = 512


def _head_body(x_ref, w_ref, b_ref, u_ref, samp_ref, slp_ref, ent_ref, kl_ref):
    n_copies = u_ref.shape[0]
    x = x_ref[...]
    w = w_ref[...]
    logits = jnp.dot(x, w, preferred_element_type=jnp.float32) + b_ref[...]
    m = jnp.max(logits, axis=-1, keepdims=True)
    sh = logits - m
    lse = jnp.log(jnp.sum(jnp.exp(sh), axis=-1, keepdims=True))
    logp = sh - lse
    p = jnp.exp(logp)
    ent_ref[0, :] = -jnp.sum(p * logp, axis=-1)
    kl_ref[0, :] = jnp.sum(p * (logp - jnp.log(1.0 / N_CAT)), axis=-1)
    iota = lax.broadcasted_iota(jnp.int32, (BM, N_CAT), 1)
    for k in range(n_copies):
        g = -jnp.log(-jnp.log(u_ref[k]))
        s = jnp.argmax(logp + g, axis=-1).astype(jnp.int32)
        samp_ref[k, :] = s
        slp_ref[k, :] = jnp.sum(jnp.where(iota == s[:, None], logp, 0.0), axis=-1)


def _head_call(xf, W, b2, u):
    bsz = xf.shape[0]
    n_copies = u.shape[0]
    grid = (bsz // BM,)
    return pl.pallas_call(
        _head_body,
        grid=grid,
        in_specs=[
            pl.BlockSpec((BM, D_IMG), lambda i: (i, 0)),
            pl.BlockSpec((D_IMG, N_CAT), lambda i: (0, 0)),
            pl.BlockSpec((1, N_CAT), lambda i: (0, 0)),
            pl.BlockSpec((n_copies, BM, N_CAT), lambda i: (0, i, 0)),
        ],
        out_specs=[
            pl.BlockSpec((n_copies, BM), lambda i: (0, i)),
            pl.BlockSpec((n_copies, BM), lambda i: (0, i)),
            pl.BlockSpec((1, BM), lambda i: (0, i)),
            pl.BlockSpec((1, BM), lambda i: (0, i)),
        ],
        out_shape=[
            jax.ShapeDtypeStruct((n_copies, bsz), jnp.int32),
            jax.ShapeDtypeStruct((n_copies, bsz), jnp.float32),
            jax.ShapeDtypeStruct((1, bsz), jnp.float32),
            jax.ShapeDtypeStruct((1, bsz), jnp.float32),
        ],
        compiler_params=pltpu.CompilerParams(
            dimension_semantics=("parallel",),
        ),
    )(xf, W, b2, u)


_SC_CH = 16


def _sc_gather_body(per_w, idx_hbm, bank_hbm, out_hbm,
                    idx_v, rows0, rows1, gs0, gs1, ss0, ss1):
    nc = 2
    wid = lax.axis_index("s") * nc + lax.axis_index("c")
    base = wid * per_w
    pltpu.sync_copy(idx_hbm.at[pl.ds(base, per_w)], idx_v)
    n_ch = per_w // _SC_CH
    bufs = ((rows0, gs0, ss0), (rows1, gs1, ss1))

    def gather(c, buf, gsem):
        pltpu.async_copy(
            bank_hbm.at[idx_v.at[pl.ds(c * _SC_CH, _SC_CH)]], buf, gsem
        )

    gather(0, rows0, gs0)
    if n_ch > 1:
        gather(1, rows1, gs1)
    for c in range(n_ch):
        buf, gsem, ssem = bufs[c % 2]
        pltpu.make_async_copy(
            bank_hbm.at[idx_v.at[pl.ds(0, _SC_CH)]], buf, gsem
        ).wait()
        st = pltpu.async_copy(
            buf, out_hbm.at[pl.ds(base + c * _SC_CH, _SC_CH)], ssem
        )
        st.wait()
        if c + 2 < n_ch:
            gather(c + 2, buf, gsem)


def _gather_call(idx, bank_flat):
    n_rows = idx.shape[0]
    info = plsc.get_sparse_core_info()
    nw = info.num_cores * info.num_subcores
    per_w = n_rows // nw
    mesh = plsc.VectorSubcoreMesh(core_axis_name="c", subcore_axis_name="s")
    kfn = pl.kernel(
        functools.partial(_sc_gather_body, per_w),
        mesh=mesh,
        out_type=jax.ShapeDtypeStruct((n_rows, D_IMG), jnp.float32),
        scratch_types=[
            pltpu.VMEM((per_w,), jnp.int32),
            pltpu.VMEM((_SC_CH, D_IMG), jnp.float32),
            pltpu.VMEM((_SC_CH, D_IMG), jnp.float32),
            pltpu.SemaphoreType.DMA,
            pltpu.SemaphoreType.DMA,
            pltpu.SemaphoreType.DMA,
            pltpu.SemaphoreType.DMA,
        ],
    )
    return kfn(idx, bank_flat)


_BF = 512
_BS = 2048


def _onehot_body(bank_t_ref, samp_ref, out_ref):
    oh = (lax.broadcasted_iota(jnp.int32, (N_CAT, _BS), 0)
          == samp_ref[...]).astype(jnp.float32)
    out_ref[...] = jnp.dot(bank_t_ref[...], oh,
                           preferred_element_type=jnp.float32)


def _onehot_gather_call(bank_t, idx_row, n_rows):
    grid = (D_IMG // _BF, n_rows // _BS)
    return pl.pallas_call(
        _onehot_body,
        grid=grid,
        in_specs=[
            pl.BlockSpec((_BF, N_CAT), lambda fi, si: (fi, 0)),
            pl.BlockSpec((1, _BS), lambda fi, si: (0, si)),
        ],
        out_specs=pl.BlockSpec((_BF, _BS), lambda fi, si: (fi, si)),
        out_shape=jax.ShapeDtypeStruct((D_IMG, n_rows), jnp.float32),
        compiler_params=pltpu.CompilerParams(
            dimension_semantics=("parallel", "parallel"),
        ),
    )(bank_t, idx_row)


def kernel(x, u, W, b, bank, n_copies):
    bsz = x.shape[0]
    n_copies_static = u.shape[0]
    n_rows = n_copies_static * bsz
    xf = x.reshape(bsz, -1)
    samp, slp, ent, kl = _head_call(xf, W, b.reshape(1, -1), u)
    idx_row = samp.reshape(1, n_rows)
    bank_t = bank.reshape(N_CAT, D_IMG).T
    out_t = _onehot_gather_call(bank_t, idx_row, n_rows)
    c, h, w = bank.shape[1:]
    x_out = jax.lax.stop_gradient(
        out_t.reshape(c, h, w, n_rows).transpose(3, 0, 1, 2)
    )
    return (x_out, slp.reshape(-1), ent.reshape(-1), kl.reshape(-1))


# speedup vs baseline: 7.0688x; 1.2073x over previous
import functools

import jax
import jax.numpy as jnp
from jax import lax
from jax.experimental import pallas as pl
from jax.experimental.pallas import tpu as pltpu
from jax.experimental.pallas import tpu_sc as plsc

N_CAT = 238
D_IMG = 3 * 32 * 32
BM = 512


BN = 512


def _mm_body(wt_ref, xt_ref, bt_ref, logpt_ref, ent_ref, kl_ref):
    wt = wt_ref[...]
    xt = xt_ref[...]
    logits = jnp.dot(wt, xt, preferred_element_type=jnp.float32) + bt_ref[...]
    m = jnp.max(logits, axis=0, keepdims=True)
    sh = logits - m
    lse = jnp.log(jnp.sum(jnp.exp(sh), axis=0, keepdims=True))
    logp = sh - lse
    p = jnp.exp(logp)
    logpt_ref[...] = logp
    ent_ref[...] = -jnp.sum(p * logp, axis=0, keepdims=True)
    kl_ref[...] = jnp.sum(p * (logp - jnp.log(1.0 / N_CAT)), axis=0,
                          keepdims=True)


def _mm_call(wt, xt, bt):
    bsz = xt.shape[1]
    grid = (bsz // BN,)
    return pl.pallas_call(
        _mm_body,
        grid=grid,
        in_specs=[
            pl.BlockSpec((N_CAT, D_IMG), lambda i: (0, 0)),
            pl.BlockSpec((D_IMG, BN), lambda i: (0, i)),
            pl.BlockSpec((N_CAT, 1), lambda i: (0, 0)),
        ],
        out_specs=[
            pl.BlockSpec((N_CAT, BN), lambda i: (0, i)),
            pl.BlockSpec((1, BN), lambda i: (0, i)),
            pl.BlockSpec((1, BN), lambda i: (0, i)),
        ],
        out_shape=[
            jax.ShapeDtypeStruct((N_CAT, bsz), jnp.float32),
            jax.ShapeDtypeStruct((1, bsz), jnp.float32),
            jax.ShapeDtypeStruct((1, bsz), jnp.float32),
        ],
        compiler_params=pltpu.CompilerParams(
            dimension_semantics=("parallel",),
        ),
    )(wt, xt, bt)


def _samp_body(logp_ref, u_ref, samp_ref, slp_ref):
    n_copies = u_ref.shape[0]
    logp = logp_ref[...]
    iota = lax.broadcasted_iota(jnp.int32, (BM, N_CAT), 1)
    for k in range(n_copies):
        g = -jnp.log(-jnp.log(u_ref[k]))
        s = jnp.argmax(logp + g, axis=-1).astype(jnp.int32)
        samp_ref[k, :] = s
        slp_ref[k, :] = jnp.sum(jnp.where(iota == s[:, None], logp, 0.0), axis=-1)


def _samp_call(logp_row, u):
    bsz = logp_row.shape[0]
    n_copies = u.shape[0]
    grid = (bsz // BM,)
    return pl.pallas_call(
        _samp_body,
        grid=grid,
        in_specs=[
            pl.BlockSpec((BM, N_CAT), lambda i: (i, 0)),
            pl.BlockSpec((n_copies, BM, N_CAT), lambda i: (0, i, 0)),
        ],
        out_specs=[
            pl.BlockSpec((n_copies, BM), lambda i: (0, i)),
            pl.BlockSpec((n_copies, BM), lambda i: (0, i)),
        ],
        out_shape=[
            jax.ShapeDtypeStruct((n_copies, bsz), jnp.int32),
            jax.ShapeDtypeStruct((n_copies, bsz), jnp.float32),
        ],
        compiler_params=pltpu.CompilerParams(
            dimension_semantics=("parallel",),
        ),
    )(logp_row, u)


_SC_CH = 16


def _sc_gather_body(per_w, idx_hbm, bank_hbm, out_hbm,
                    idx_v, rows0, rows1, gs0, gs1, ss0, ss1):
    nc = 2
    wid = lax.axis_index("s") * nc + lax.axis_index("c")
    base = wid * per_w
    pltpu.sync_copy(idx_hbm.at[pl.ds(base, per_w)], idx_v)
    n_ch = per_w // _SC_CH
    bufs = ((rows0, gs0, ss0), (rows1, gs1, ss1))

    def gather(c, buf, gsem):
        pltpu.async_copy(
            bank_hbm.at[idx_v.at[pl.ds(c * _SC_CH, _SC_CH)]], buf, gsem
        )

    gather(0, rows0, gs0)
    if n_ch > 1:
        gather(1, rows1, gs1)
    for c in range(n_ch):
        buf, gsem, ssem = bufs[c % 2]
        pltpu.make_async_copy(
            bank_hbm.at[idx_v.at[pl.ds(0, _SC_CH)]], buf, gsem
        ).wait()
        st = pltpu.async_copy(
            buf, out_hbm.at[pl.ds(base + c * _SC_CH, _SC_CH)], ssem
        )
        st.wait()
        if c + 2 < n_ch:
            gather(c + 2, buf, gsem)


def _gather_call(idx, bank_flat):
    n_rows = idx.shape[0]
    info = plsc.get_sparse_core_info()
    nw = info.num_cores * info.num_subcores
    per_w = n_rows // nw
    mesh = plsc.VectorSubcoreMesh(core_axis_name="c", subcore_axis_name="s")
    kfn = pl.kernel(
        functools.partial(_sc_gather_body, per_w),
        mesh=mesh,
        out_type=jax.ShapeDtypeStruct((n_rows, D_IMG), jnp.float32),
        scratch_types=[
            pltpu.VMEM((per_w,), jnp.int32),
            pltpu.VMEM((_SC_CH, D_IMG), jnp.float32),
            pltpu.VMEM((_SC_CH, D_IMG), jnp.float32),
            pltpu.SemaphoreType.DMA,
            pltpu.SemaphoreType.DMA,
            pltpu.SemaphoreType.DMA,
            pltpu.SemaphoreType.DMA,
        ],
    )
    return kfn(idx, bank_flat)


_BF = 512
_BS = 2048


def _onehot_body(bank_t_ref, samp_ref, out_ref):
    oh = (lax.broadcasted_iota(jnp.int32, (N_CAT, _BS), 0)
          == samp_ref[...]).astype(jnp.float32)
    out_ref[...] = jnp.dot(bank_t_ref[...], oh,
                           preferred_element_type=jnp.float32)


def _onehot_gather_call(bank_t, idx_row, n_rows):
    grid = (D_IMG // _BF, n_rows // _BS)
    return pl.pallas_call(
        _onehot_body,
        grid=grid,
        in_specs=[
            pl.BlockSpec((_BF, N_CAT), lambda fi, si: (fi, 0)),
            pl.BlockSpec((1, _BS), lambda fi, si: (0, si)),
        ],
        out_specs=pl.BlockSpec((_BF, _BS), lambda fi, si: (fi, si)),
        out_shape=jax.ShapeDtypeStruct((D_IMG, n_rows), jnp.float32),
        compiler_params=pltpu.CompilerParams(
            dimension_semantics=("parallel", "parallel"),
        ),
    )(bank_t, idx_row)


def kernel(x, u, W, b, bank, n_copies):
    bsz = x.shape[0]
    n_copies_static = u.shape[0]
    n_rows = n_copies_static * bsz
    xt = x.reshape(bsz, -1).T
    logpt, ent, kl = _mm_call(W.T, xt, b.reshape(-1, 1))
    samp, slp = _samp_call(logpt.T, u)
    idx_row = samp.reshape(1, n_rows)
    bank_t = bank.reshape(N_CAT, D_IMG).T
    out_t = _onehot_gather_call(bank_t, idx_row, n_rows)
    c, h, w = bank.shape[1:]
    x_out = jax.lax.stop_gradient(
        out_t.reshape(c, h, w, n_rows).transpose(3, 0, 1, 2)
    )
    return (x_out, slp.reshape(-1), ent.reshape(-1), kl.reshape(-1))
